# baseline (speedup 1.0000x reference)
import functools

import jax
import jax.numpy as jnp
from jax import lax
from jax.experimental import pallas as pl
from jax.experimental.pallas import tpu as pltpu
from jax.experimental.pallas import tpu_sc as plsc

P = 0.05
_LANES = 16
_NTILES = 32


def _drop16(base):
    n = base.astype(jnp.uint32) + lax.iota(jnp.uint32, 16)
    k1 = jnp.uint32(0)
    k2 = jnp.uint32(42)
    ks0, ks1, ks2 = k1, k2, k1 ^ k2 ^ jnp.uint32(0x1BD11BDA)
    rots = ((13, 15, 26, 6), (17, 29, 16, 24))
    kseq = ((ks1, ks2), (ks2, ks0), (ks0, ks1), (ks1, ks2), (ks2, ks0))
    x0 = jnp.zeros((16,), jnp.uint32) + ks0
    x1 = n + ks1
    for i in range(5):
        for r in rots[i % 2]:
            x0 = x0 + x1
            x1 = (x1 << jnp.uint32(r)) | (x1 >> jnp.uint32(32 - r))
            x1 = x0 ^ x1
        ka, kb = kseq[i]
        x0 = x0 + ka
        x1 = x1 + kb + jnp.uint32(i + 1)
    bits = x0 ^ x1
    return (bits >> jnp.uint32(9)) < jnp.uint32(419431)


@functools.partial(jax.jit, static_argnames=("pad", "tpw"))
def _node_drop_masks(m, *, pad, tpw):
    mesh = plsc.VectorSubcoreMesh(core_axis_name="c", subcore_axis_name="s")

    @functools.partial(
        pl.kernel,
        mesh=mesh,
        out_type=jax.ShapeDtypeStruct((3 * pad,), jnp.int32),
        scratch_types=[pltpu.VMEM((3 * tpw,), jnp.int32)],
    )
    def body(m_hbm, out_hbm, buf):
        wid = lax.axis_index("s") * 2 + lax.axis_index("c")
        base = pl.multiple_of(wid * tpw, 8)
        for k in range(3):
            pltpu.sync_copy(
                m_hbm.at[pl.ds(base + k * pad, tpw)],
                buf.at[pl.ds(k * tpw, tpw)],
            )
        zero = jnp.zeros((16,), jnp.int32)

        def chunk(c, carry):
            off = c * _LANES
            drop = _drop16(base + off)
            for k in range(3):
                sl = pl.ds(k * tpw + off, _LANES)
                buf[sl] = jnp.where(drop, zero, buf[sl])
            return carry

        lax.fori_loop(0, tpw // _LANES, chunk, 0)
        for k in range(3):
            pltpu.sync_copy(
                buf.at[pl.ds(k * tpw, tpw)],
                out_hbm.at[pl.ds(base + k * pad, tpw)],
            )

    return body(m)


def _passthrough_copies(x, e, y):
    def body(x_in, e_in, y_in, x_out, e_out, y_out, sx, se, sy):
        cx = pltpu.make_async_copy(x_in, x_out, sx)
        ce = pltpu.make_async_copy(e_in, e_out, se)
        cy = pltpu.make_async_copy(y_in, y_out, sy)
        cx.start()
        ce.start()
        cy.start()
        cx.wait()
        ce.wait()
        cy.wait()

    return pl.pallas_call(
        body,
        in_specs=[pl.BlockSpec(memory_space=pltpu.MemorySpace.HBM)] * 3,
        out_specs=[pl.BlockSpec(memory_space=pltpu.MemorySpace.HBM)] * 3,
        out_shape=[
            jax.ShapeDtypeStruct(x.shape, x.dtype),
            jax.ShapeDtypeStruct(e.shape, e.dtype),
            jax.ShapeDtypeStruct(y.shape, y.dtype),
        ],
        scratch_shapes=[pltpu.SemaphoreType.DMA] * 3,
    )(x, e, y)


def kernel(x, edge_index, y, train_mask, test_mask, val_mask):
    n = train_mask.shape[0]
    chunk = _NTILES * _LANES
    pad = ((n + chunk - 1) // chunk) * chunk
    tpw = pad // _NTILES
    m = jnp.concatenate(
        [
            jnp.pad(train_mask.astype(jnp.int32), (0, pad - n)),
            jnp.pad(test_mask.astype(jnp.int32), (0, pad - n)),
            jnp.pad(val_mask.astype(jnp.int32), (0, pad - n)),
        ]
    )
    out = _node_drop_masks(m, pad=pad, tpw=tpw)
    x_out, e_out, y_out = _passthrough_copies(x, edge_index, y)
    new_train = out[0:n].astype(jnp.bool_)
    new_test = out[pad:pad + n].astype(jnp.bool_)
    new_val = out[2 * pad:2 * pad + n].astype(jnp.bool_)
    return (x_out, e_out, y_out, new_train, new_val, new_test)


# speedup vs baseline: 7.1637x; 7.1637x over previous
import functools

import jax
import jax.numpy as jnp
from jax import lax
from jax.experimental import pallas as pl
from jax.experimental.pallas import tpu as pltpu
from jax.experimental.pallas import tpu_sc as plsc

P = 0.05
_LANES = 16
_NTILES = 32


def _drop16(base):
    n = base.astype(jnp.uint32) + lax.iota(jnp.uint32, 16)
    k1 = jnp.uint32(0)
    k2 = jnp.uint32(42)
    ks0, ks1, ks2 = k1, k2, k1 ^ k2 ^ jnp.uint32(0x1BD11BDA)
    rots = ((13, 15, 26, 6), (17, 29, 16, 24))
    kseq = ((ks1, ks2), (ks2, ks0), (ks0, ks1), (ks1, ks2), (ks2, ks0))
    x0 = jnp.zeros((16,), jnp.uint32) + ks0
    x1 = n + ks1
    for i in range(5):
        for r in rots[i % 2]:
            x0 = x0 + x1
            x1 = (x1 << jnp.uint32(r)) | (x1 >> jnp.uint32(32 - r))
            x1 = x0 ^ x1
        ka, kb = kseq[i]
        x0 = x0 + ka
        x1 = x1 + kb + jnp.uint32(i + 1)
    bits = x0 ^ x1
    return (bits >> jnp.uint32(9)) < jnp.uint32(419431)


@functools.partial(jax.jit, static_argnames=("pad", "tpw"))
def _node_drop_masks(m, *, pad, tpw):
    mesh = plsc.VectorSubcoreMesh(core_axis_name="c", subcore_axis_name="s")

    @functools.partial(
        pl.kernel,
        mesh=mesh,
        out_type=jax.ShapeDtypeStruct((3 * pad,), jnp.int32),
        scratch_types=[pltpu.VMEM((3 * tpw,), jnp.int32)],
    )
    def body(m_hbm, out_hbm, buf):
        wid = lax.axis_index("s") * 2 + lax.axis_index("c")
        base = pl.multiple_of(wid * tpw, 8)
        for k in range(3):
            pltpu.sync_copy(
                m_hbm.at[pl.ds(base + k * pad, tpw)],
                buf.at[pl.ds(k * tpw, tpw)],
            )
        zero = jnp.zeros((16,), jnp.int32)

        def chunk(c, carry):
            off = c * _LANES
            drop = _drop16(base + off)
            for k in range(3):
                sl = pl.ds(k * tpw + off, _LANES)
                buf[sl] = jnp.where(drop, zero, buf[sl])
            return carry

        lax.fori_loop(0, tpw // _LANES, chunk, 0)
        for k in range(3):
            pltpu.sync_copy(
                buf.at[pl.ds(k * tpw, tpw)],
                out_hbm.at[pl.ds(base + k * pad, tpw)],
            )

    return body(m)


def _passthrough_copies(x, e, y):
    def body(x_in, e_in, y_in, x_out, e_out, y_out):
        x_out[...] = x_in[...]
        e_out[...] = e_in[...]
        y_out[...] = y_in[...]

    steps = 25
    return pl.pallas_call(
        body,
        grid=(steps,),
        in_specs=[
            pl.BlockSpec((x.shape[0] // steps, x.shape[1]), lambda i: (i, 0)),
            pl.BlockSpec((2, e.shape[1] // steps), lambda i: (0, i)),
            pl.BlockSpec((y.shape[0],), lambda i: (0,)),
        ],
        out_specs=[
            pl.BlockSpec((x.shape[0] // steps, x.shape[1]), lambda i: (i, 0)),
            pl.BlockSpec((2, e.shape[1] // steps), lambda i: (0, i)),
            pl.BlockSpec((y.shape[0],), lambda i: (0,)),
        ],
        out_shape=[
            jax.ShapeDtypeStruct(x.shape, x.dtype),
            jax.ShapeDtypeStruct(e.shape, e.dtype),
            jax.ShapeDtypeStruct(y.shape, y.dtype),
        ],
    )(x, e, y)


def kernel(x, edge_index, y, train_mask, test_mask, val_mask):
    n = train_mask.shape[0]
    chunk = _NTILES * _LANES
    pad = ((n + chunk - 1) // chunk) * chunk
    tpw = pad // _NTILES
    m = jnp.concatenate(
        [
            jnp.pad(train_mask.astype(jnp.int32), (0, pad - n)),
            jnp.pad(test_mask.astype(jnp.int32), (0, pad - n)),
            jnp.pad(val_mask.astype(jnp.int32), (0, pad - n)),
        ]
    )
    out = _node_drop_masks(m, pad=pad, tpw=tpw)
    x_out, e_out, y_out = _passthrough_copies(x, edge_index, y)
    new_train = out[0:n].astype(jnp.bool_)
    new_test = out[pad:pad + n].astype(jnp.bool_)
    new_val = out[2 * pad:2 * pad + n].astype(jnp.bool_)
    return (x_out, e_out, y_out, new_train, new_val, new_test)


# speedup vs baseline: 7.6264x; 1.0646x over previous
import functools

import jax
import jax.numpy as jnp
from jax import lax
from jax.experimental import pallas as pl
from jax.experimental.pallas import tpu as pltpu
from jax.experimental.pallas import tpu_sc as plsc

P = 0.05
_LANES = 16
_NTILES = 32


def _drop16(n):
    k1 = jnp.uint32(0)
    k2 = jnp.uint32(42)
    ks0, ks1, ks2 = k1, k2, k1 ^ k2 ^ jnp.uint32(0x1BD11BDA)
    rots = ((13, 15, 26, 6), (17, 29, 16, 24))
    kseq = ((ks1, ks2), (ks2, ks0), (ks0, ks1), (ks1, ks2), (ks2, ks0))
    x0 = jnp.zeros((16,), jnp.uint32) + ks0
    x1 = n + ks1
    for i in range(5):
        for r in rots[i % 2]:
            x0 = x0 + x1
            x1 = (x1 << jnp.uint32(r)) | (x1 >> jnp.uint32(32 - r))
            x1 = x0 ^ x1
        ka, kb = kseq[i]
        x0 = x0 + ka
        x1 = x1 + kb + jnp.uint32(i + 1)
    bits = x0 ^ x1
    return (bits >> jnp.uint32(9)) < jnp.uint32(419431)


@functools.partial(jax.jit, static_argnames=("n", "tpw"))
def _node_drop_masks(t, te, v, *, n, tpw):
    mesh = plsc.VectorSubcoreMesh(core_axis_name="c", subcore_axis_name="s")
    shape = jax.ShapeDtypeStruct((n,), jnp.int32)

    @functools.partial(
        pl.kernel,
        mesh=mesh,
        out_type=(shape, shape, shape),
        scratch_types=[pltpu.VMEM((3 * tpw,), jnp.int32)],
    )
    def body(t_hbm, te_hbm, v_hbm, to_hbm, teo_hbm, vo_hbm, buf):
        wid = lax.axis_index("s") * 2 + lax.axis_index("c")
        base = pl.multiple_of(jnp.minimum(wid * tpw, n - tpw), 8)
        for k, ref in enumerate((t_hbm, te_hbm, v_hbm)):
            pltpu.sync_copy(ref.at[pl.ds(base, tpw)], buf.at[pl.ds(k * tpw, tpw)])
        zero = jnp.zeros((16,), jnp.int32)

        def chunk(c, carry):
            off = c * _LANES
            nvec = (base + off).astype(jnp.uint32) + lax.iota(jnp.uint32, 16)
            drop = _drop16(nvec)
            for k in range(3):
                sl = pl.ds(k * tpw + off, _LANES)
                buf[sl] = jnp.where(drop, zero, buf[sl])
            return carry

        lax.fori_loop(0, tpw // _LANES, chunk, 0)
        for k, ref in enumerate((to_hbm, teo_hbm, vo_hbm)):
            pltpu.sync_copy(buf.at[pl.ds(k * tpw, tpw)], ref.at[pl.ds(base, tpw)])

    return body(t, te, v)


def kernel(x, edge_index, y, train_mask, test_mask, val_mask):
    n = train_mask.shape[0]
    tpw = -(-n // (_NTILES * _LANES)) * _LANES
    t, te, v = _node_drop_masks(
        train_mask.astype(jnp.int32),
        test_mask.astype(jnp.int32),
        val_mask.astype(jnp.int32),
        n=n,
        tpw=tpw,
    )
    return (
        x,
        edge_index,
        y,
        t.astype(jnp.bool_),
        v.astype(jnp.bool_),
        te.astype(jnp.bool_),
    )


# speedup vs baseline: 8.3533x; 1.0953x over previous
import functools

import jax
import jax.numpy as jnp
from jax import lax
from jax.experimental import pallas as pl
from jax.experimental.pallas import tpu as pltpu
from jax.experimental.pallas import tpu_sc as plsc

P = 0.05
_LANES = 16
_NCORES = 1
_NSUB = 16
_NTILES = _NCORES * _NSUB


def _drop16(n):
    k1 = jnp.uint32(0)
    k2 = jnp.uint32(42)
    ks0, ks1, ks2 = k1, k2, k1 ^ k2 ^ jnp.uint32(0x1BD11BDA)
    rots = ((13, 15, 26, 6), (17, 29, 16, 24))
    kseq = ((ks1, ks2), (ks2, ks0), (ks0, ks1), (ks1, ks2), (ks2, ks0))
    x0 = jnp.zeros((16,), jnp.uint32) + ks0
    x1 = n + ks1
    for i in range(5):
        for r in rots[i % 2]:
            x0 = x0 + x1
            x1 = (x1 << jnp.uint32(r)) | (x1 >> jnp.uint32(32 - r))
            x1 = x0 ^ x1
        ka, kb = kseq[i]
        x0 = x0 + ka
        x1 = x1 + kb + jnp.uint32(i + 1)
    bits = x0 ^ x1
    return (bits >> jnp.uint32(9)) < jnp.uint32(419431)


@functools.partial(jax.jit, static_argnames=("n", "tpw"))
def _node_drop_masks(m, *, n, tpw):
    mesh = plsc.VectorSubcoreMesh(
        core_axis_name="c", subcore_axis_name="s", num_cores=_NCORES
    )

    @functools.partial(
        pl.kernel,
        mesh=mesh,
        out_type=jax.ShapeDtypeStruct((3 * n,), jnp.int32),
        scratch_types=[pltpu.VMEM((3 * tpw,), jnp.int32)],
    )
    def body(m_hbm, o_hbm, buf):
        wid = lax.axis_index("s") * _NCORES + lax.axis_index("c")
        base = pl.multiple_of(jnp.minimum(wid * tpw, n - tpw), 8)
        for k in range(3):
            pltpu.sync_copy(
                m_hbm.at[pl.ds(k * n + base, tpw)], buf.at[pl.ds(k * tpw, tpw)]
            )
        zero = jnp.zeros((16,), jnp.int32)

        def chunk(c, carry):
            off = c * _LANES
            nvec = (base + off).astype(jnp.uint32) + lax.iota(jnp.uint32, 16)
            drop = _drop16(nvec)
            for k in range(3):
                sl = pl.ds(k * tpw + off, _LANES)
                buf[sl] = jnp.where(drop, zero, buf[sl])
            return carry

        lax.fori_loop(0, tpw // _LANES, chunk, 0)
        for k in range(3):
            pltpu.sync_copy(
                buf.at[pl.ds(k * tpw, tpw)], o_hbm.at[pl.ds(k * n + base, tpw)]
            )

    return body(m)


def kernel(x, edge_index, y, train_mask, test_mask, val_mask):
    n = train_mask.shape[0]
    chunk = _NTILES * _LANES
    tpw = (-(-n // chunk)) * _LANES
    m = jnp.concatenate([train_mask, test_mask, val_mask]).astype(jnp.int32)
    out = _node_drop_masks(m, n=n, tpw=tpw).astype(jnp.bool_)
    return (x, edge_index, y, out[0:n], out[2 * n:3 * n], out[n:2 * n])
